# baseline (speedup 1.0000x reference)
import functools

import jax
import jax.numpy as jnp
from jax import lax
from jax.experimental import pallas as pl
from jax.experimental.pallas import tpu as pltpu
from jax.experimental.pallas import tpu_sc as plsc

K = 2
LAMBDA_MAX = 0.1
DELTA = 0.5

NC = 2
NS = 16
L = 16

NEG = -3.0e38


def _rsqrt(x):
    xi = plsc.bitcast(x, jnp.int32)
    yi = jnp.int32(0x5F3759DF) - lax.shift_right_logical(xi, 1)
    y = plsc.bitcast(yi, jnp.float32)
    for _ in range(4):
        y = y * (1.5 - 0.5 * x * y * y)
    return y


def _sc_body(b_n, n, d, feat, w_hbm, out,
             rows_v, w_v, a_pub, inv_v, mu_v, krow1_v, krow2_v, part_v, red_v,
             loss_v, accA_buf, accQ_buf, dsems, sh_top, sh_part):
    rpt = n // NS
    ng = rpt // L
    bags = b_n // NC
    c = lax.axis_index("c")
    s = lax.axis_index("s")
    lane = lax.iota(jnp.int32, L)

    pltpu.sync_copy(w_hbm, w_v)

    def bag_body(j, carry):
        b = c * bags + j
        row0 = s * rpt
        nrc = 4
        rc_rows = rpt // nrc
        cps = [
            pltpu.async_copy(
                feat.at[b, pl.ds(row0 + i * rc_rows, rc_rows)],
                rows_v.at[pl.ds(i * rc_rows, rc_rows)],
                dsems[i],
            )
            for i in range(nrc)
        ]

        zero = jnp.zeros((L,), jnp.float32)
        nblk = 4
        bcols = d // nblk
        bch = bcols // L
        for rc in range(nrc):
            cps[rc].wait()
            for blk in range(nblk):
                wvs = [w_v[pl.ds(blk * bcols + kk * L, L)] for kk in range(bch)]
                def row_body(r, _, blk=blk, wvs=wvs):
                    rbase = blk * bcols
                    aa = [zero] * 2
                    qq = [zero] * 2
                    for kk in range(bch):
                        fv = rows_v[r, pl.ds(rbase + kk * L, L)]
                        aa[kk % 2] = aa[kk % 2] + fv * wvs[kk]
                        qq[kk % 2] = qq[kk % 2] + fv * fv
                    pa = aa[0] + aa[1]
                    pq = qq[0] + qq[1]
                    if blk > 0:
                        pa = pa + accA_buf[pl.ds(r * L, L)]
                        pq = pq + accQ_buf[pl.ds(r * L, L)]
                    accA_buf[pl.ds(r * L, L)] = pa
                    accQ_buf[pl.ds(r * L, L)] = pq
                    return 0
                lax.fori_loop(rc * rc_rows, (rc + 1) * rc_rows, row_body, 0)
        for g in range(ng):
            gbase = (g * L + lane) * L
            ta = [zero] * 4
            tq = [zero] * 4
            for l in range(16):
                ta[l % 4] = ta[l % 4] + plsc.load_gather(accA_buf, [gbase + l])
                tq[l % 4] = tq[l % 4] + plsc.load_gather(accQ_buf, [gbase + l])
            acc_a = (ta[0] + ta[1]) + (ta[2] + ta[3])
            acc_q = (tq[0] + tq[1]) + (tq[2] + tq[3])
            a_pub[pl.ds(g * L, L)] = acc_a
            inv_v[pl.ds(g * L, L)] = _rsqrt(jnp.maximum(acc_q, 1e-24))

        def insert3(carry, v, g):
            v1, x1, v2, x2, v3, x3 = carry
            b1 = v > v1
            nv1 = jnp.where(b1, v, v1)
            nx1 = jnp.where(b1, g, x1)
            dv = jnp.where(b1, v1, v)
            dx = jnp.where(b1, x1, g)
            b2 = dv > v2
            nv2 = jnp.where(b2, dv, v2)
            nx2 = jnp.where(b2, dx, x2)
            dv2 = jnp.where(b2, v2, dv)
            dx2 = jnp.where(b2, x2, dx)
            b3 = dv2 > v3
            nv3 = jnp.where(b3, dv2, v3)
            nx3 = jnp.where(b3, dx2, x3)
            return (nv1, nx1, nv2, nx2, nv3, nx3)

        def merge3(v1, x1, v2, x2, v3):
            m1 = jnp.max(v1)
            i1 = jnp.min(jnp.where(v1 == m1, x1, n))
            c1 = jnp.where(x1 == i1, NEG, v1)
            m2 = jnp.maximum(jnp.max(c1), jnp.max(v2))
            i2a = jnp.min(jnp.where(c1 == m2, x1, n))
            i2b = jnp.min(jnp.where(v2 == m2, x2, n))
            i2 = jnp.minimum(i2a, i2b)
            c1b = jnp.where(x1 == i2, NEG, c1)
            c2 = jnp.where(x2 == i2, NEG, v2)
            m3 = jnp.maximum(jnp.maximum(jnp.max(c1b), jnp.max(c2)),
                             jnp.max(v3))
            return m1, i1, m2, i2, m3

        negs = jnp.full((L,), NEG)
        zi = jnp.zeros((L,), jnp.int32)
        st = (negs, zi, negs, zi, negs, zi)
        for ci in range(ng):
            st = insert3(st, a_pub[pl.ds(ci * L, L)], row0 + ci * L + lane)
        lm1, li1, lm2, li2, lm3 = merge3(st[0], st[1], st[2], st[3], st[4])

        valv = jnp.where(lane == 0, lm1,
                         jnp.where(lane == 1, lm2,
                                   jnp.where(lane == 2, lm3, 0.0)))
        xv = jnp.where(lane == 3, li1, jnp.where(lane == 4, li2, 0))
        part_v[...] = jnp.where(lane < 3, valv,
                                plsc.bitcast(xv, jnp.float32))
        pltpu.sync_copy(part_v, sh_top.at[pl.ds(s * L, L)])
        plsc.subcore_barrier()
        pltpu.sync_copy(sh_top, red_v)

        gl = lane * L
        tv1 = plsc.load_gather(red_v, [gl])
        tv2 = plsc.load_gather(red_v, [gl + 1])
        tv3 = plsc.load_gather(red_v, [gl + 2])
        tx1 = plsc.bitcast(plsc.load_gather(red_v, [gl + 3]), jnp.int32)
        tx2 = plsc.bitcast(plsc.load_gather(red_v, [gl + 4]), jnp.int32)
        _, i1, _, i2, m3 = merge3(tv1, tx1, tv2, tx2, tv3)

        kcp1 = pltpu.async_copy(feat.at[b, pl.ds(i1, 1)], krow1_v, dsems[0])
        kcp2 = pltpu.async_copy(feat.at[b, pl.ds(i2, 1)], krow2_v, dsems[1])
        kcp1.wait()
        kcp2.wait()

        def mu_body(o, acc):
            for k in range(8):
                dcol = (o * 8 + k) * L
                m = (krow1_v[0, pl.ds(dcol, L)] + krow2_v[0, pl.ds(dcol, L)]) * 0.5
                mu_v[pl.ds(dcol, L)] = m
                acc = acc + m * m
            return acc
        acc = lax.fori_loop(0, d // (8 * L), mu_body, jnp.zeros((L,), jnp.float32))
        ssq = jnp.sum(acc)
        rmu = jnp.max(_rsqrt(jnp.maximum(jnp.broadcast_to(ssq, (L,)), 1e-24)))

        se = jnp.float32(0.0)
        sed = jnp.float32(0.0)
        dmax = NEG
        for blk in range(nblk):
            mvs = [mu_v[pl.ds(blk * bcols + kk * L, L)] for kk in range(bch)]
            def row_body2(r, _, blk=blk, mvs=mvs):
                rbase = blk * bcols
                dd = [zero] * 2
                for kk in range(bch):
                    fv = rows_v[r, pl.ds(rbase + kk * L, L)]
                    dd[kk % 2] = dd[kk % 2] + fv * mvs[kk]
                pd = dd[0] + dd[1]
                if blk > 0:
                    pd = pd + accA_buf[pl.ds(r * L, L)]
                accA_buf[pl.ds(r * L, L)] = pd
                return 0
            lax.fori_loop(0, rpt, row_body2, 0)
        for g in range(ng):
            gbase = (g * L + lane) * L
            td = [zero] * 4
            for l in range(16):
                td[l % 4] = td[l % 4] + plsc.load_gather(accA_buf, [gbase + l])
            acc_d = (td[0] + td[1]) + (td[2] + td[3])
            dvec = 1.0 - acc_d * inv_v[pl.ds(g * L, L)] * rmu
            gvec = row0 + g * L + lane
            km = (gvec == i1) | (gvec == i2)
            avec = a_pub[pl.ds(g * L, L)]
            e = jnp.where(km, 0.0, jnp.exp(avec - m3))
            se = se + jnp.sum(e)
            sed = sed + jnp.sum(e * dvec)
            dmax = jnp.maximum(dmax, jnp.max(jnp.where(km, NEG, dvec)))

        part = jnp.where(lane == 0, se,
                         jnp.where(lane == 1, sed,
                                   jnp.where(lane == 2, dmax, 0.0)))
        part_v[...] = part
        pltpu.sync_copy(part_v, sh_part.at[pl.ds(s * L, L)])
        plsc.subcore_barrier()

        @pl.when(s == 0)
        def _():
            pltpu.sync_copy(sh_part, red_v)
            accv = jnp.zeros((L,), jnp.float32)
            accm = jnp.full((L,), NEG)
            for t in range(NS):
                vt = red_v[pl.ds(t * L, L)]
                accv = accv + vt
                accm = jnp.maximum(accm, vt)
            s_e = jnp.sum(jnp.where(lane == 0, accv, 0.0))
            s_ed = jnp.sum(jnp.where(lane == 1, accv, 0.0))
            dmx = jnp.sum(jnp.where(lane == 2, accm, 0.0))
            s_e_v = jnp.broadcast_to(s_e, (L,))
            s_ed_v = jnp.broadcast_to(s_ed, (L,))
            dmx_v = jnp.broadcast_to(dmx, (L,))
            loss_v[...] = (s_ed_v / s_e_v
                           + LAMBDA_MAX * jnp.maximum(dmx_v - DELTA, 0.0))
            pltpu.sync_copy(loss_v, out.at[b])

        return carry

    lax.fori_loop(0, bags, bag_body, jnp.int32(0))


def kernel(features, labels, head_w):
    del labels
    b, n, d = features.shape
    rpt = n // NS
    w = head_w.reshape(d)

    body = functools.partial(_sc_body, b, n, d)
    sc = pl.kernel(
        body,
        out_type=jax.ShapeDtypeStruct((b, L), jnp.float32),
        mesh=plsc.VectorSubcoreMesh(
            core_axis_name="c", subcore_axis_name="s",
            num_cores=NC, num_subcores=NS,
        ),
        scratch_types=[
            pltpu.VMEM((rpt, d), jnp.float32),
            pltpu.VMEM((d,), jnp.float32),
            pltpu.VMEM((rpt,), jnp.float32),
            pltpu.VMEM((rpt,), jnp.float32),
            pltpu.VMEM((d,), jnp.float32),
            pltpu.VMEM((1, d), jnp.float32),
            pltpu.VMEM((1, d), jnp.float32),
            pltpu.VMEM((L,), jnp.float32),
            pltpu.VMEM((NS * L,), jnp.float32),
            pltpu.VMEM((L,), jnp.float32),
            pltpu.VMEM((rpt * L,), jnp.float32),
            pltpu.VMEM((rpt * L,), jnp.float32),
            [pltpu.SemaphoreType.DMA] * 4,
            pltpu.VMEM_SHARED((NS * L,), jnp.float32),
            pltpu.VMEM_SHARED((NS * L,), jnp.float32),
        ],
        compiler_params=pltpu.CompilerParams(needs_layout_passes=False),
    )
    out = sc(features, w)
    return jnp.mean(out[:, 0])


# speedup vs baseline: 1.1248x; 1.1248x over previous
import functools

import jax
import jax.numpy as jnp
from jax import lax
from jax.experimental import pallas as pl
from jax.experimental.pallas import tpu as pltpu
from jax.experimental.pallas import tpu_sc as plsc

K = 2
LAMBDA_MAX = 0.1
DELTA = 0.5

NC = 2
NS = 16
L = 16

NEG = -3.0e38


def _rsqrt(x):
    xi = plsc.bitcast(x, jnp.int32)
    yi = jnp.int32(0x5F3759DF) - lax.shift_right_logical(xi, 1)
    y = plsc.bitcast(yi, jnp.float32)
    for _ in range(4):
        y = y * (1.5 - 0.5 * x * y * y)
    return y


def _sc_body(b_n, n, d, feat, w_hbm, out,
             rows_v, w_v, a_pub, inv_v, mu_v, krow1_v, krow2_v, part_v, red_v,
             loss_v, accA_buf, accQ_buf, dsems, sh_top, sh_part):
    rpt = n // NS
    ng = rpt // L
    bags = b_n // NC
    c = lax.axis_index("c")
    s = lax.axis_index("s")
    lane = lax.iota(jnp.int32, L)

    pltpu.sync_copy(w_hbm, w_v)

    def bag_body(j, carry):
        b = c * bags + j
        row0 = s * rpt
        nrc = 4
        rc_rows = rpt // nrc
        cps = [
            pltpu.async_copy(
                feat.at[b, pl.ds(row0 + i * rc_rows, rc_rows)],
                rows_v.at[pl.ds(i * rc_rows, rc_rows)],
                dsems[i],
            )
            for i in range(nrc)
        ]

        zero = jnp.zeros((L,), jnp.float32)
        nblk = 2
        bcols = d // nblk
        bch = bcols // L
        for rc in range(nrc):
            cps[rc].wait()
            for blk in range(nblk):
                wvs = [w_v[pl.ds(blk * bcols + kk * L, L)] for kk in range(bch)]
                def row_body(r, _, blk=blk, wvs=wvs):
                    rbase = blk * bcols
                    aa = [zero] * 2
                    qq = [zero] * 2
                    for kk in range(bch):
                        fv = rows_v[r, pl.ds(rbase + kk * L, L)]
                        aa[kk % 2] = aa[kk % 2] + fv * wvs[kk]
                        qq[kk % 2] = qq[kk % 2] + fv * fv
                    pa = aa[0] + aa[1]
                    pq = qq[0] + qq[1]
                    if blk > 0:
                        pa = pa + accA_buf[pl.ds(r * L, L)]
                        pq = pq + accQ_buf[pl.ds(r * L, L)]
                    accA_buf[pl.ds(r * L, L)] = pa
                    accQ_buf[pl.ds(r * L, L)] = pq
                    return 0
                lax.fori_loop(rc * rc_rows, (rc + 1) * rc_rows, row_body, 0)
        for g in range(ng):
            gbase = (g * L + lane) * L
            ta = [zero] * 4
            tq = [zero] * 4
            for l in range(16):
                ta[l % 4] = ta[l % 4] + plsc.load_gather(accA_buf, [gbase + l])
                tq[l % 4] = tq[l % 4] + plsc.load_gather(accQ_buf, [gbase + l])
            acc_a = (ta[0] + ta[1]) + (ta[2] + ta[3])
            acc_q = (tq[0] + tq[1]) + (tq[2] + tq[3])
            a_pub[pl.ds(g * L, L)] = acc_a
            inv_v[pl.ds(g * L, L)] = _rsqrt(jnp.maximum(acc_q, 1e-24))

        def insert3(carry, v, g):
            v1, x1, v2, x2, v3, x3 = carry
            b1 = v > v1
            nv1 = jnp.where(b1, v, v1)
            nx1 = jnp.where(b1, g, x1)
            dv = jnp.where(b1, v1, v)
            dx = jnp.where(b1, x1, g)
            b2 = dv > v2
            nv2 = jnp.where(b2, dv, v2)
            nx2 = jnp.where(b2, dx, x2)
            dv2 = jnp.where(b2, v2, dv)
            dx2 = jnp.where(b2, x2, dx)
            b3 = dv2 > v3
            nv3 = jnp.where(b3, dv2, v3)
            nx3 = jnp.where(b3, dx2, x3)
            return (nv1, nx1, nv2, nx2, nv3, nx3)

        def merge3(v1, x1, v2, x2, v3):
            m1 = jnp.max(v1)
            i1 = jnp.min(jnp.where(v1 == m1, x1, n))
            c1 = jnp.where(x1 == i1, NEG, v1)
            m2 = jnp.maximum(jnp.max(c1), jnp.max(v2))
            i2a = jnp.min(jnp.where(c1 == m2, x1, n))
            i2b = jnp.min(jnp.where(v2 == m2, x2, n))
            i2 = jnp.minimum(i2a, i2b)
            c1b = jnp.where(x1 == i2, NEG, c1)
            c2 = jnp.where(x2 == i2, NEG, v2)
            m3 = jnp.maximum(jnp.maximum(jnp.max(c1b), jnp.max(c2)),
                             jnp.max(v3))
            return m1, i1, m2, i2, m3

        negs = jnp.full((L,), NEG)
        zi = jnp.zeros((L,), jnp.int32)
        st = (negs, zi, negs, zi, negs, zi)
        for ci in range(ng):
            st = insert3(st, a_pub[pl.ds(ci * L, L)], row0 + ci * L + lane)
        lm1, li1, lm2, li2, lm3 = merge3(st[0], st[1], st[2], st[3], st[4])

        valv = jnp.where(lane == 0, lm1,
                         jnp.where(lane == 1, lm2,
                                   jnp.where(lane == 2, lm3, 0.0)))
        xv = jnp.where(lane == 3, li1, jnp.where(lane == 4, li2, 0))
        part_v[...] = jnp.where(lane < 3, valv,
                                plsc.bitcast(xv, jnp.float32))
        pltpu.sync_copy(part_v, sh_top.at[pl.ds(s * L, L)])
        plsc.subcore_barrier()
        pltpu.sync_copy(sh_top, red_v)

        gl = lane * L
        tv1 = plsc.load_gather(red_v, [gl])
        tv2 = plsc.load_gather(red_v, [gl + 1])
        tv3 = plsc.load_gather(red_v, [gl + 2])
        tx1 = plsc.bitcast(plsc.load_gather(red_v, [gl + 3]), jnp.int32)
        tx2 = plsc.bitcast(plsc.load_gather(red_v, [gl + 4]), jnp.int32)
        _, i1, _, i2, m3 = merge3(tv1, tx1, tv2, tx2, tv3)

        kcp1 = pltpu.async_copy(feat.at[b, pl.ds(i1, 1)], krow1_v, dsems[0])
        kcp2 = pltpu.async_copy(feat.at[b, pl.ds(i2, 1)], krow2_v, dsems[1])
        kcp1.wait()
        kcp2.wait()

        def mu_body(o, acc):
            for k in range(8):
                dcol = (o * 8 + k) * L
                m = (krow1_v[0, pl.ds(dcol, L)] + krow2_v[0, pl.ds(dcol, L)]) * 0.5
                mu_v[pl.ds(dcol, L)] = m
                acc = acc + m * m
            return acc
        acc = lax.fori_loop(0, d // (8 * L), mu_body, jnp.zeros((L,), jnp.float32))
        ssq = jnp.sum(acc)
        rmu = jnp.max(_rsqrt(jnp.maximum(jnp.broadcast_to(ssq, (L,)), 1e-24)))

        se = jnp.float32(0.0)
        sed = jnp.float32(0.0)
        dmax = NEG
        for blk in range(nblk):
            mvs = [mu_v[pl.ds(blk * bcols + kk * L, L)] for kk in range(bch)]
            def row_body2(r, _, blk=blk, mvs=mvs):
                rbase = blk * bcols
                dd = [zero] * 2
                for kk in range(bch):
                    fv = rows_v[r, pl.ds(rbase + kk * L, L)]
                    dd[kk % 2] = dd[kk % 2] + fv * mvs[kk]
                pd = dd[0] + dd[1]
                if blk > 0:
                    pd = pd + accA_buf[pl.ds(r * L, L)]
                accA_buf[pl.ds(r * L, L)] = pd
                return 0
            lax.fori_loop(0, rpt, row_body2, 0)
        for g in range(ng):
            gbase = (g * L + lane) * L
            td = [zero] * 4
            for l in range(16):
                td[l % 4] = td[l % 4] + plsc.load_gather(accA_buf, [gbase + l])
            acc_d = (td[0] + td[1]) + (td[2] + td[3])
            dvec = 1.0 - acc_d * inv_v[pl.ds(g * L, L)] * rmu
            gvec = row0 + g * L + lane
            km = (gvec == i1) | (gvec == i2)
            avec = a_pub[pl.ds(g * L, L)]
            e = jnp.where(km, 0.0, jnp.exp(avec - m3))
            se = se + jnp.sum(e)
            sed = sed + jnp.sum(e * dvec)
            dmax = jnp.maximum(dmax, jnp.max(jnp.where(km, NEG, dvec)))

        part = jnp.where(lane == 0, se,
                         jnp.where(lane == 1, sed,
                                   jnp.where(lane == 2, dmax, 0.0)))
        part_v[...] = part
        pltpu.sync_copy(part_v, sh_part.at[pl.ds(s * L, L)])
        plsc.subcore_barrier()

        @pl.when(s == 0)
        def _():
            pltpu.sync_copy(sh_part, red_v)
            accv = jnp.zeros((L,), jnp.float32)
            accm = jnp.full((L,), NEG)
            for t in range(NS):
                vt = red_v[pl.ds(t * L, L)]
                accv = accv + vt
                accm = jnp.maximum(accm, vt)
            s_e = jnp.sum(jnp.where(lane == 0, accv, 0.0))
            s_ed = jnp.sum(jnp.where(lane == 1, accv, 0.0))
            dmx = jnp.sum(jnp.where(lane == 2, accm, 0.0))
            s_e_v = jnp.broadcast_to(s_e, (L,))
            s_ed_v = jnp.broadcast_to(s_ed, (L,))
            dmx_v = jnp.broadcast_to(dmx, (L,))
            loss_v[...] = (s_ed_v / s_e_v
                           + LAMBDA_MAX * jnp.maximum(dmx_v - DELTA, 0.0))
            pltpu.sync_copy(loss_v, out.at[b])

        return carry

    lax.fori_loop(0, bags, bag_body, jnp.int32(0))


def kernel(features, labels, head_w):
    del labels
    b, n, d = features.shape
    rpt = n // NS
    w = head_w.reshape(d)

    body = functools.partial(_sc_body, b, n, d)
    sc = pl.kernel(
        body,
        out_type=jax.ShapeDtypeStruct((b, L), jnp.float32),
        mesh=plsc.VectorSubcoreMesh(
            core_axis_name="c", subcore_axis_name="s",
            num_cores=NC, num_subcores=NS,
        ),
        scratch_types=[
            pltpu.VMEM((rpt, d), jnp.float32),
            pltpu.VMEM((d,), jnp.float32),
            pltpu.VMEM((rpt,), jnp.float32),
            pltpu.VMEM((rpt,), jnp.float32),
            pltpu.VMEM((d,), jnp.float32),
            pltpu.VMEM((1, d), jnp.float32),
            pltpu.VMEM((1, d), jnp.float32),
            pltpu.VMEM((L,), jnp.float32),
            pltpu.VMEM((NS * L,), jnp.float32),
            pltpu.VMEM((L,), jnp.float32),
            pltpu.VMEM((rpt * L,), jnp.float32),
            pltpu.VMEM((rpt * L,), jnp.float32),
            [pltpu.SemaphoreType.DMA] * 4,
            pltpu.VMEM_SHARED((NS * L,), jnp.float32),
            pltpu.VMEM_SHARED((NS * L,), jnp.float32),
        ],
        compiler_params=pltpu.CompilerParams(needs_layout_passes=False),
    )
    out = sc(features, w)
    return jnp.mean(out[:, 0])
